# baseline (speedup 1.0000x reference)
import functools

import jax
import jax.numpy as jnp
from jax import lax
from jax.experimental import pallas as pl
from jax.experimental.pallas import tpu as pltpu
from jax.experimental.pallas import tpu_sc as plsc

_NC = 2
_NS = 16
_NW = _NC * _NS
_LANES = 16

_GATHER = 128
_BLKS = 8
_CHUNK = _GATHER * _BLKS


def _rsqrt16(x):
    i = plsc.bitcast(x, jnp.int32)
    i = jnp.int32(0x5F3759DF) - lax.shift_right_logical(i, 1)
    y = plsc.bitcast(i, jnp.float32)
    nh = x * jnp.float32(-0.5)
    for _ in range(3):
        y = y * (jnp.float32(1.5) + nh * y * y)
    return y


def _make_sc_kernel(n_rows, d):
    assert d == 4 * _LANES
    per_w = n_rows // _NW
    assert per_w * _NW == n_rows
    n_chunks = per_w // _CHUNK
    assert n_chunks * _CHUNK == per_w
    mesh = plsc.VectorSubcoreMesh(
        core_axis_name="c", subcore_axis_name="s",
        num_cores=_NC, num_subcores=_NS)

    @functools.partial(
        pl.kernel,
        out_type=jax.ShapeDtypeStruct((n_rows, d), jnp.float32),
        mesh=mesh,
        compiler_params=pltpu.CompilerParams(
            needs_layout_passes=False, use_tc_tiling_on_sc=False),
        scratch_types=[
            pltpu.VMEM((_BLKS, _GATHER), jnp.int32),
            pltpu.VMEM((_CHUNK, d), jnp.float32),
            pltpu.VMEM((d,), jnp.float32),
            pltpu.VMEM((d,), jnp.float32),
            pltpu.SemaphoreType.DMA,
        ],
    )
    def sc_kernel(x_hbm, table_hbm, gamma_hbm, beta_hbm, out_hbm,
                  idx_v, rows_v, g_v, b_v, sem):
        wid = lax.axis_index("s") * _NC + lax.axis_index("c")
        pltpu.sync_copy(gamma_hbm, g_v)
        pltpu.sync_copy(beta_hbm, b_v)
        g = [g_v[pl.ds(k * _LANES, _LANES)] for k in range(4)]
        b = [b_v[pl.ds(k * _LANES, _LANES)] for k in range(4)]
        base_blk = wid * (per_w // _GATHER)

        @pl.loop(0, n_chunks)
        def _chunk(c):
            blk0 = base_blk + c * _BLKS
            pltpu.sync_copy(x_hbm.at[pl.ds(blk0, _BLKS)], idx_v)
            for j in range(_BLKS):
                pltpu.async_copy(
                    table_hbm.at[idx_v.at[j]],
                    rows_v.at[pl.ds(j * _GATHER, _GATHER)], sem)
            for j in range(_BLKS):
                pltpu.make_async_copy(
                    table_hbm.at[idx_v.at[j]],
                    rows_v.at[pl.ds(j * _GATHER, _GATHER)], sem).wait()

            @pl.loop(0, _CHUNK)
            def _row(r):
                v = [rows_v[r, pl.ds(k * _LANES, _LANES)] for k in range(4)]
                s = (v[0] + v[1]) + (v[2] + v[3])
                q = (v[0] * v[0] + v[1] * v[1]) + (v[2] * v[2] + v[3] * v[3])
                mean = jnp.sum(s) * jnp.float32(1.0 / 64.0)
                ex2 = jnp.sum(q) * jnp.float32(1.0 / 64.0)
                var = ex2 - mean * mean
                vvar = jnp.full((_LANES,), var + jnp.float32(1e-5), jnp.float32)
                rstd = _rsqrt16(vvar)
                vmean = jnp.full((_LANES,), mean, jnp.float32)
                for k in range(4):
                    a = rstd * g[k]
                    cc = b[k] - vmean * a
                    rows_v[r, pl.ds(k * _LANES, _LANES)] = v[k] * a + cc

            pltpu.sync_copy(
                rows_v, out_hbm.at[pl.ds(blk0 * _GATHER, _CHUNK)])

    return sc_kernel


def kernel(x, table, gamma, beta):
    bsz, seq = x.shape
    d = table.shape[1]
    n = bsz * seq
    x2 = x.reshape(n // _GATHER, _GATHER).astype(jnp.int32)
    out = _make_sc_kernel(n, d)(x2, table, gamma, beta)
    return out.reshape(bsz, seq, d)
